# Initial kernel scaffold; baseline (speedup 1.0000x reference)
import jax
import jax.numpy as jnp
from jax.experimental import pallas as pl


def kernel(center_ids, pos_ctx_ids, neg_ctx_ids, ent_targets, center_emb, context_emb, W, b):
    raise NotImplementedError("write your pallas kernel here")


# speedup vs baseline: 6.1869x; 6.1869x over previous
import functools

import jax
import jax.numpy as jnp
from jax import lax
from jax.experimental import pallas as pl
from jax.experimental.pallas import tpu as pltpu
from jax.experimental.pallas import tpu_sc as plsc

SEM_DIM = 64
ENT_DIM = 64

_NC = 2
_NS = 16
_NW = _NC * _NS

_RC = 64


def _sc_gather(center_emb, context_emb, center_ids, pos_ids, neg_ids_flat, K):
    B = center_ids.shape[0]
    DC = center_emb.shape[1]
    D = context_emb.shape[1]
    b_w = B // _NW
    n_ch = b_w // _RC

    mesh = plsc.VectorSubcoreMesh(
        core_axis_name="c", subcore_axis_name="s",
        num_cores=_NC, num_subcores=_NS)

    @functools.partial(
        pl.kernel,
        out_type=(
            jax.ShapeDtypeStruct((B, DC), jnp.float32),
            jax.ShapeDtypeStruct((B, D), jnp.float32),
            jax.ShapeDtypeStruct((K, B, D), jnp.float32),
        ),
        mesh=mesh,
        scratch_types=[
            pltpu.VMEM((_RC,), jnp.int32),
            pltpu.VMEM((_RC,), jnp.int32),
            pltpu.VMEM((K, _RC), jnp.int32),
            pltpu.VMEM((_RC, DC), jnp.float32),
            pltpu.VMEM((_RC, D), jnp.float32),
            pltpu.VMEM((K, _RC, D), jnp.float32),
            pltpu.SemaphoreType.DMA,
            pltpu.SemaphoreType.DMA,
        ],
        compiler_params=pltpu.CompilerParams(use_tc_tiling_on_sc=False),
    )
    def k(cemb, xemb, cids, pids, nids, cout, pout, nout,
          cidx, pidx, nidx, crows, prows, nrows, sem_g, sem_w):
        wid = lax.axis_index("s") * _NC + lax.axis_index("c")
        for ch in range(n_ch):
            base = wid * b_w + ch * _RC
            pltpu.sync_copy(cids.at[pl.ds(base, _RC)], cidx)
            pltpu.sync_copy(pids.at[pl.ds(base, _RC)], pidx)
            for kk in range(K):
                pltpu.sync_copy(nids.at[pl.ds(kk * B + base, _RC)], nidx.at[kk])
            copies = [
                pltpu.async_copy(cemb.at[cidx], crows, sem_g),
                pltpu.async_copy(xemb.at[pidx], prows, sem_g),
            ]
            for kk in range(K):
                copies.append(
                    pltpu.async_copy(xemb.at[nidx.at[kk]], nrows.at[kk], sem_g))
            for c in copies:
                c.wait()
            writes = [
                pltpu.async_copy(crows, cout.at[pl.ds(base, _RC)], sem_w),
                pltpu.async_copy(prows, pout.at[pl.ds(base, _RC)], sem_w),
                pltpu.async_copy(nrows, nout.at[:, pl.ds(base, _RC)], sem_w),
            ]
            for c in writes:
                c.wait()

    return k(center_emb, context_emb, center_ids, pos_ids, neg_ids_flat)


def _tc_body(cref, pref, nref, tref, wtref, bref, oref):
    i = pl.program_id(0)
    c = cref[...]
    sem = c[:, :SEM_DIM]
    ent = c[:, SEM_DIM:]
    pos = pref[...]
    neg = nref[...]
    ps = jnp.sum(sem * pos, axis=1)
    ns = jnp.sum(neg * sem[None], axis=2)

    def log_sigmoid(x):
        return jnp.minimum(x, 0.0) - jnp.log(1.0 + jnp.exp(-jnp.abs(x)))

    s_skip = jnp.sum(log_sigmoid(ps)) + jnp.sum(log_sigmoid(-ns))
    pred = jnp.dot(ent, wtref[...], preferred_element_type=jnp.float32)
    pred = pred + bref[...]
    s_ent = jnp.sum((pred - tref[...]) ** 2)
    lane = lax.broadcasted_iota(jnp.int32, (1, 128), 1)
    v = jnp.where(lane == 0, s_skip, jnp.where(lane == 1, s_ent, 0.0))

    @pl.when(i == 0)
    def _():
        oref[...] = v

    @pl.when(i > 0)
    def _():
        oref[...] += v


def _tc_compute(center_all, pos_ctx, neg_ctx, ent_targets, w_t, b2):
    B = center_all.shape[0]
    K = neg_ctx.shape[0]
    R = 512
    nb = B // R
    return pl.pallas_call(
        _tc_body,
        grid=(nb,),
        in_specs=[
            pl.BlockSpec((R, 2 * SEM_DIM), lambda i: (i, 0)),
            pl.BlockSpec((R, SEM_DIM), lambda i: (i, 0)),
            pl.BlockSpec((K, R, SEM_DIM), lambda i: (0, i, 0)),
            pl.BlockSpec((R, ENT_DIM), lambda i: (i, 0)),
            pl.BlockSpec((ENT_DIM, ENT_DIM), lambda i: (0, 0)),
            pl.BlockSpec((1, ENT_DIM), lambda i: (0, 0)),
        ],
        out_specs=pl.BlockSpec((1, 128), lambda i: (0, 0)),
        out_shape=jax.ShapeDtypeStruct((1, 128), jnp.float32),
    )(center_all, pos_ctx, neg_ctx, ent_targets, w_t, b2)


def kernel(center_ids, pos_ctx_ids, neg_ctx_ids, ent_targets,
           center_emb, context_emb, W, b):
    B = center_ids.shape[0]
    K = neg_ctx_ids.shape[1]
    cids = center_ids.astype(jnp.int32)
    pids = pos_ctx_ids.astype(jnp.int32)
    nids_flat = neg_ctx_ids.astype(jnp.int32).T.reshape(K * B)
    center_all, pos_ctx, neg_ctx = _sc_gather(
        center_emb, context_emb, cids, pids, nids_flat, K)
    sums = _tc_compute(center_all, pos_ctx, neg_ctx, ent_targets,
                       W.T, b.reshape(1, ENT_DIM))
    skipgram_loss = -sums[0, 0] / B
    ent_loss = sums[0, 1] / (B * ENT_DIM)
    return (skipgram_loss + ent_loss, skipgram_loss, ent_loss)


# speedup vs baseline: 6.3465x; 1.0258x over previous
import functools

import jax
import jax.numpy as jnp
from jax import lax
from jax.experimental import pallas as pl
from jax.experimental.pallas import tpu as pltpu
from jax.experimental.pallas import tpu_sc as plsc

SEM_DIM = 64
ENT_DIM = 64

_NC = 2
_NS = 16
_NW = _NC * _NS

_RC = 64
_GI = 128


def _sc_gather(center_emb, context_emb, center_ids, pos_ids, neg_ids_flat, K):
    B = center_ids.shape[0]
    DC = center_emb.shape[1]
    D = context_emb.shape[1]
    b_w = B // _NW
    n_ch = b_w // _RC
    ng = _RC * K // _GI

    mesh = plsc.VectorSubcoreMesh(
        core_axis_name="c", subcore_axis_name="s",
        num_cores=_NC, num_subcores=_NS)

    @functools.partial(
        pl.kernel,
        out_type=(
            jax.ShapeDtypeStruct((B, DC), jnp.float32),
            jax.ShapeDtypeStruct((B, D), jnp.float32),
            jax.ShapeDtypeStruct((B * K, D), jnp.float32),
        ),
        mesh=mesh,
        scratch_types=[
            pltpu.VMEM((_RC,), jnp.int32),
            pltpu.VMEM((_RC,), jnp.int32),
            pltpu.VMEM((_RC * K,), jnp.int32),
            pltpu.VMEM((_RC, DC), jnp.float32),
            pltpu.VMEM((_RC, D), jnp.float32),
            pltpu.VMEM((_RC * K, D), jnp.float32),
            pltpu.SemaphoreType.DMA,
            pltpu.SemaphoreType.DMA,
        ],
        compiler_params=pltpu.CompilerParams(use_tc_tiling_on_sc=False),
    )
    def k(cemb, xemb, cids, pids, nids, cout, pout, nout,
          cidx, pidx, nidx, crows, prows, nrows, sem_g, sem_w):
        wid = lax.axis_index("s") * _NC + lax.axis_index("c")
        for ch in range(n_ch):
            base = wid * b_w + ch * _RC
            nbase = base * K
            pltpu.sync_copy(cids.at[pl.ds(base, _RC)], cidx)
            pltpu.sync_copy(pids.at[pl.ds(base, _RC)], pidx)
            pltpu.sync_copy(nids.at[pl.ds(nbase, _RC * K)], nidx)
            copies = [
                pltpu.async_copy(cemb.at[cidx], crows, sem_g),
                pltpu.async_copy(xemb.at[pidx], prows, sem_g),
            ]
            for g in range(ng):
                copies.append(pltpu.async_copy(
                    xemb.at[nidx.at[pl.ds(g * _GI, _GI)]],
                    nrows.at[pl.ds(g * _GI, _GI)], sem_g))
            for c in copies:
                c.wait()
            writes = [
                pltpu.async_copy(crows, cout.at[pl.ds(base, _RC)], sem_w),
                pltpu.async_copy(prows, pout.at[pl.ds(base, _RC)], sem_w),
                pltpu.async_copy(nrows, nout.at[pl.ds(nbase, _RC * K)], sem_w),
            ]
            for c in writes:
                c.wait()

    return k(center_emb, context_emb, center_ids, pos_ids, neg_ids_flat)


def _tc_body(cref, pref, nref, tref, wtref, bref, oref):
    i = pl.program_id(0)
    c = cref[...]
    sem = c[:, :SEM_DIM]
    ent = c[:, SEM_DIM:]
    pos = pref[...]
    neg = nref[...]
    ps = jnp.sum(sem * pos, axis=1)
    ns = jnp.sum(neg * sem[:, None, :], axis=2)

    def log_sigmoid(x):
        return jnp.minimum(x, 0.0) - jnp.log(1.0 + jnp.exp(-jnp.abs(x)))

    s_skip = jnp.sum(log_sigmoid(ps)) + jnp.sum(log_sigmoid(-ns))
    pred = jnp.dot(ent, wtref[...], preferred_element_type=jnp.float32)
    pred = pred + bref[...]
    s_ent = jnp.sum((pred - tref[...]) ** 2)
    lane = lax.broadcasted_iota(jnp.int32, (1, 128), 1)
    v = jnp.where(lane == 0, s_skip, jnp.where(lane == 1, s_ent, 0.0))

    @pl.when(i == 0)
    def _():
        oref[...] = v

    @pl.when(i > 0)
    def _():
        oref[...] += v


def _tc_compute(center_all, pos_ctx, neg_ctx, ent_targets, w_t, b2):
    B = center_all.shape[0]
    K = neg_ctx.shape[1]
    R = 512
    nb = B // R
    return pl.pallas_call(
        _tc_body,
        grid=(nb,),
        in_specs=[
            pl.BlockSpec((R, 2 * SEM_DIM), lambda i: (i, 0)),
            pl.BlockSpec((R, SEM_DIM), lambda i: (i, 0)),
            pl.BlockSpec((R, K, SEM_DIM), lambda i: (i, 0, 0)),
            pl.BlockSpec((R, ENT_DIM), lambda i: (i, 0)),
            pl.BlockSpec((ENT_DIM, ENT_DIM), lambda i: (0, 0)),
            pl.BlockSpec((1, ENT_DIM), lambda i: (0, 0)),
        ],
        out_specs=pl.BlockSpec((1, 128), lambda i: (0, 0)),
        out_shape=jax.ShapeDtypeStruct((1, 128), jnp.float32),
    )(center_all, pos_ctx, neg_ctx, ent_targets, w_t, b2)


def kernel(center_ids, pos_ctx_ids, neg_ctx_ids, ent_targets,
           center_emb, context_emb, W, b):
    B = center_ids.shape[0]
    K = neg_ctx_ids.shape[1]
    cids = center_ids.astype(jnp.int32)
    pids = pos_ctx_ids.astype(jnp.int32)
    nids_flat = neg_ctx_ids.astype(jnp.int32).reshape(B * K)
    center_all, pos_ctx, neg_flat = _sc_gather(
        center_emb, context_emb, cids, pids, nids_flat, K)
    neg_ctx = neg_flat.reshape(B, K, SEM_DIM)
    sums = _tc_compute(center_all, pos_ctx, neg_ctx, ent_targets,
                       W.T, b.reshape(1, ENT_DIM))
    skipgram_loss = -sums[0, 0] / B
    ent_loss = sums[0, 1] / (B * ENT_DIM)
    return (skipgram_loss + ent_loss, skipgram_loss, ent_loss)


# speedup vs baseline: 6.3730x; 1.0042x over previous
import functools

import jax
import jax.numpy as jnp
from jax import lax
from jax.experimental import pallas as pl
from jax.experimental.pallas import tpu as pltpu
from jax.experimental.pallas import tpu_sc as plsc

SEM_DIM = 64
ENT_DIM = 64

_NC = 2
_NS = 16
_NW = _NC * _NS

_RC = 64
_GI = 128


def _sc_gather(center_emb, context_emb, center_ids, pos_ids, neg_ids_flat, K):
    B = center_ids.shape[0]
    DC = center_emb.shape[1]
    D = context_emb.shape[1]
    b_w = B // _NW
    n_ch = b_w // _RC
    ng = _RC * K // _GI

    mesh = plsc.VectorSubcoreMesh(
        core_axis_name="c", subcore_axis_name="s",
        num_cores=_NC, num_subcores=_NS)

    @functools.partial(
        pl.kernel,
        out_type=(
            jax.ShapeDtypeStruct((B, DC), jnp.float32),
            jax.ShapeDtypeStruct((B, D), jnp.float32),
            jax.ShapeDtypeStruct((B * K, D), jnp.float32),
        ),
        mesh=mesh,
        scratch_types=[
            pltpu.VMEM((_RC,), jnp.int32),
            pltpu.VMEM((_RC,), jnp.int32),
            pltpu.VMEM((_RC * K,), jnp.int32),
            pltpu.VMEM((_RC, DC), jnp.float32),
            pltpu.VMEM((_RC, D), jnp.float32),
            pltpu.VMEM((_RC * K, D), jnp.float32),
            pltpu.SemaphoreType.DMA,
            pltpu.SemaphoreType.DMA,
        ],
        compiler_params=pltpu.CompilerParams(use_tc_tiling_on_sc=False),
    )
    def k(cemb, xemb, cids, pids, nids, cout, pout, nout,
          cidx, pidx, nidx, crows, prows, nrows, sem_g, sem_w):
        wid = lax.axis_index("s") * _NC + lax.axis_index("c")
        for ch in range(n_ch):
            base = wid * b_w + ch * _RC
            nbase = base * K
            pltpu.sync_copy(cids.at[pl.ds(base, _RC)], cidx)
            pltpu.sync_copy(pids.at[pl.ds(base, _RC)], pidx)
            pltpu.sync_copy(nids.at[pl.ds(nbase, _RC * K)], nidx)
            copies = [
                pltpu.async_copy(cemb.at[cidx], crows, sem_g),
                pltpu.async_copy(xemb.at[pidx], prows, sem_g),
            ]
            for g in range(ng):
                copies.append(pltpu.async_copy(
                    xemb.at[nidx.at[pl.ds(g * _GI, _GI)]],
                    nrows.at[pl.ds(g * _GI, _GI)], sem_g))
            for c in copies:
                c.wait()
            writes = [
                pltpu.async_copy(crows, cout.at[pl.ds(base, _RC)], sem_w),
                pltpu.async_copy(prows, pout.at[pl.ds(base, _RC)], sem_w),
                pltpu.async_copy(nrows, nout.at[pl.ds(nbase, _RC * K)], sem_w),
            ]
            for c in writes:
                c.wait()

    return k(center_emb, context_emb, center_ids, pos_ids, neg_ids_flat)


def _tc_body(cref, pref, nref, tref, wtref, bref, oref):
    i = pl.program_id(0)
    c = cref[...]
    sem = c[:, :SEM_DIM]
    ent = c[:, SEM_DIM:]
    pos = pref[...]
    negf = nref[...]
    neg = negf.reshape(sem.shape[0], -1, SEM_DIM)
    ps = jnp.sum(sem * pos, axis=1)
    ns = jnp.sum(neg * sem[:, None, :], axis=2)

    def log_sigmoid(x):
        return jnp.minimum(x, 0.0) - jnp.log(1.0 + jnp.exp(-jnp.abs(x)))

    s_skip = jnp.sum(log_sigmoid(ps)) + jnp.sum(log_sigmoid(-ns))
    pred = jnp.dot(ent, wtref[...], preferred_element_type=jnp.float32)
    pred = pred + bref[...]
    s_ent = jnp.sum((pred - tref[...]) ** 2)
    lane = lax.broadcasted_iota(jnp.int32, (1, 128), 1)
    v = jnp.where(lane == 0, s_skip, jnp.where(lane == 1, s_ent, 0.0))

    @pl.when(i == 0)
    def _():
        oref[...] = v

    @pl.when(i > 0)
    def _():
        oref[...] += v


def _tc_compute(center_all, pos_ctx, neg_ctx, ent_targets, w_t, b2):
    B = center_all.shape[0]
    K = neg_ctx.shape[0] // B
    R = 512
    nb = B // R
    return pl.pallas_call(
        _tc_body,
        grid=(nb,),
        in_specs=[
            pl.BlockSpec((R, 2 * SEM_DIM), lambda i: (i, 0)),
            pl.BlockSpec((R, SEM_DIM), lambda i: (i, 0)),
            pl.BlockSpec((R * K, SEM_DIM), lambda i: (i, 0)),
            pl.BlockSpec((R, ENT_DIM), lambda i: (i, 0)),
            pl.BlockSpec((ENT_DIM, ENT_DIM), lambda i: (0, 0)),
            pl.BlockSpec((1, ENT_DIM), lambda i: (0, 0)),
        ],
        out_specs=pl.BlockSpec((1, 128), lambda i: (0, 0)),
        out_shape=jax.ShapeDtypeStruct((1, 128), jnp.float32),
    )(center_all, pos_ctx, neg_ctx, ent_targets, w_t, b2)


def kernel(center_ids, pos_ctx_ids, neg_ctx_ids, ent_targets,
           center_emb, context_emb, W, b):
    B = center_ids.shape[0]
    K = neg_ctx_ids.shape[1]
    cids = center_ids.astype(jnp.int32)
    pids = pos_ctx_ids.astype(jnp.int32)
    nids_flat = neg_ctx_ids.astype(jnp.int32).reshape(B * K)
    center_all, pos_ctx, neg_flat = _sc_gather(
        center_emb, context_emb, cids, pids, nids_flat, K)
    sums = _tc_compute(center_all, pos_ctx, neg_flat, ent_targets,
                       W.T, b.reshape(1, ENT_DIM))
    skipgram_loss = -sums[0, 0] / B
    ent_loss = sums[0, 1] / (B * ENT_DIM)
    return (skipgram_loss + ent_loss, skipgram_loss, ent_loss)
